# baseline (speedup 1.0000x reference)
import functools

import jax
import jax.numpy as jnp
from jax import lax
from jax.experimental import pallas as pl
from jax.experimental.pallas import tpu as pltpu
from jax.experimental.pallas import tpu_sc as plsc

N = 10000
E = 320000
D = 128
L = 16
NC = 2
NS = 16
NW = NC * NS
K = 96
CHUNKS = 105
E_PER_W = K * CHUNKS
EPAD = E_PER_W * NW
RPT = 624
RPT_LAST = N - (NS - 1) * RPT
NBUF = 3


def _sc_partials(features, rows, cols, values):
    mesh = plsc.VectorSubcoreMesh(
        core_axis_name="c", subcore_axis_name="s", num_cores=NC, num_subcores=NS
    )

    @functools.partial(
        pl.kernel,
        out_type=jax.ShapeDtypeStruct((NC, N, D), jnp.float32),
        mesh=mesh,
        scratch_types=[
            pltpu.VMEM((E_PER_W,), jnp.int32),
            [pltpu.VMEM((K,), jnp.int32) for _ in range(NBUF)],
            [pltpu.VMEM((K,), jnp.float32) for _ in range(NBUF)],
            [pltpu.VMEM((K, D), jnp.float32) for _ in range(NBUF)],
            [pltpu.SemaphoreType.DMA for _ in range(NBUF)],
            [pltpu.SemaphoreType.DMA for _ in range(NBUF)],
            [pltpu.SemaphoreType.DMA for _ in range(NBUF)],
            [pltpu.SemaphoreType.DMA for _ in range(NBUF)],
            pltpu.VMEM_SHARED((N, D), jnp.float32),
        ],
    )
    def k(feat_hbm, row_hbm, col_hbm, val_hbm, out_hbm,
          cols_v, rowb, valb, gath, gsem, rsem, vsem, ssem, acc_sh):
        cid = lax.axis_index("c")
        sid = lax.axis_index("s")
        base_row = sid * RPT

        def zrow(i, carry):
            for d in range(D // L):
                gath[0][i, pl.ds(d * L, L)] = jnp.zeros((L,), jnp.float32)
            return carry
        lax.fori_loop(0, K, zrow, 0)
        for z in range(RPT // K):
            pltpu.sync_copy(gath[0], acc_sh.at[pl.ds(base_row + z * K, K)])
        nfull = (RPT // K) * K

        @pl.when(sid < NS - 1)
        def _():
            pltpu.sync_copy(gath[0].at[pl.ds(0, RPT - nfull)],
                            acc_sh.at[pl.ds(base_row + nfull, RPT - nfull)])

        @pl.when(sid == NS - 1)
        def _():
            pltpu.sync_copy(gath[0].at[pl.ds(0, RPT_LAST - nfull)],
                            acc_sh.at[pl.ds(base_row + nfull, RPT_LAST - nfull)])
        plsc.subcore_barrier()

        wid = cid * NS + sid
        ebase = wid * E_PER_W
        pltpu.sync_copy(col_hbm.at[pl.ds(ebase, E_PER_W)], cols_v)

        def start(kk, b):
            pltpu.async_copy(
                feat_hbm.at[cols_v.at[pl.ds(kk * K, K)]], gath[b], gsem[b])
            pltpu.async_copy(row_hbm.at[pl.ds(ebase + kk * K, K)],
                             rowb[b], rsem[b])
            pltpu.async_copy(val_hbm.at[pl.ds(ebase + kk * K, K)],
                             valb[b], vsem[b])

        def drain_scatter(b):
            pltpu.make_async_copy(gath[b], acc_sh.at[rowb[b]], ssem[b]).wait()

        def process(kk, b):
            pltpu.make_async_copy(row_hbm.at[pl.ds(ebase + kk * K, K)],
                                  rowb[b], rsem[b]).wait()
            pltpu.make_async_copy(val_hbm.at[pl.ds(ebase + kk * K, K)],
                                  valb[b], vsem[b]).wait()
            pltpu.make_async_copy(
                feat_hbm.at[cols_v.at[pl.ds(kk * K, K)]], gath[b],
                gsem[b]).wait()

            def scale(g, c2):
                v16 = valb[b][pl.ds(g * L, L)]
                for j in range(L):
                    v = v16[j]
                    r = g * L + j
                    for d in range(D // L):
                        sl = pl.ds(d * L, L)
                        gath[b][r, sl] = gath[b][r, sl] * v
                return c2
            lax.fori_loop(0, K // L, scale, 0)
            pltpu.async_copy(gath[b], acc_sh.at[rowb[b]], ssem[b], add=True)

        start(0, 0)
        start(1, 1)

        def chunk(kk, carry):
            for b in range(NBUF):
                @pl.when(kk % NBUF == b)
                def _(b=b):
                    process(kk, b)
                    bn = (b + 2) % NBUF

                    @pl.when(kk >= 1)
                    def _():
                        drain_scatter(bn)

                    @pl.when(kk + 2 < CHUNKS)
                    def _():
                        start(kk + 2, bn)
            return carry
        lax.fori_loop(0, CHUNKS, chunk, 0)
        drain_scatter((CHUNKS - 1) % NBUF)

        plsc.subcore_barrier()
        @pl.when(sid < NS - 1)
        def _():
            pltpu.sync_copy(acc_sh.at[pl.ds(base_row, RPT)],
                            out_hbm.at[cid, pl.ds(base_row, RPT)])

        @pl.when(sid == NS - 1)
        def _():
            pltpu.sync_copy(acc_sh.at[pl.ds(base_row, RPT_LAST)],
                            out_hbm.at[cid, pl.ds(base_row, RPT_LAST)])

    return k(features, rows, cols, values)


def _tc_sum(partials):
    RB = 400

    def body(p_ref, o_ref):
        o_ref[...] = p_ref[0] + p_ref[1]

    return pl.pallas_call(
        body,
        grid=(N // RB,),
        in_specs=[pl.BlockSpec((2, RB, D), lambda i: (0, i, 0))],
        out_specs=pl.BlockSpec((RB, D), lambda i: (i, 0)),
        out_shape=jax.ShapeDtypeStruct((N, D), jnp.float32),
    )(partials)


def kernel(features, laplacianMat_indices, laplacianMat_values, selfLoop):
    del selfLoop
    pad = EPAD - E
    rows = jnp.concatenate(
        [laplacianMat_indices[0], jnp.zeros((pad,), jnp.int32)])
    cols = jnp.concatenate(
        [laplacianMat_indices[1], jnp.zeros((pad,), jnp.int32)])
    vals = jnp.concatenate(
        [laplacianMat_values, jnp.zeros((pad,), jnp.float32)])
    partials = _sc_partials(features, rows, cols, vals)
    return _tc_sum(partials)


# speedup vs baseline: 1.7499x; 1.7499x over previous
import functools

import jax
import jax.numpy as jnp
from jax import lax
from jax.experimental import pallas as pl
from jax.experimental.pallas import tpu as pltpu
from jax.experimental.pallas import tpu_sc as plsc

N = 10000
E = 320000
D = 128
L = 16
NC = 2
NS = 16
NW = NC * NS
E_PER_W = E // NW
K = 80
CHUNKS = E_PER_W // K
RPT = 624
RPT_LAST = N - (NS - 1) * RPT
NBUF = 3


def _sc_partials(features, rows, cols, values):
    mesh = plsc.VectorSubcoreMesh(
        core_axis_name="c", subcore_axis_name="s", num_cores=NC, num_subcores=NS
    )

    @functools.partial(
        pl.kernel,
        out_type=jax.ShapeDtypeStruct((NC, N, D), jnp.float32),
        mesh=mesh,
        scratch_types=[
            pltpu.VMEM((E_PER_W,), jnp.int32),
            [pltpu.VMEM((K,), jnp.int32) for _ in range(NBUF)],
            [pltpu.VMEM((K,), jnp.float32) for _ in range(NBUF)],
            [pltpu.VMEM((K, D), jnp.float32) for _ in range(NBUF)],
            [pltpu.SemaphoreType.DMA for _ in range(NBUF)],
            [pltpu.SemaphoreType.DMA for _ in range(NBUF)],
            [pltpu.SemaphoreType.DMA for _ in range(NBUF)],
            [pltpu.SemaphoreType.DMA for _ in range(NBUF)],
            pltpu.VMEM_SHARED((N, D), jnp.float32),
        ],
    )
    def k(feat_hbm, row_hbm, col_hbm, val_hbm, out_hbm,
          cols_v, rowb, valb, gath, gsem, rsem, vsem, ssem, acc_sh):
        cid = lax.axis_index("c")
        sid = lax.axis_index("s")
        base_row = sid * RPT

        def zrow(i, carry):
            for d in range(D // L):
                gath[0][i, pl.ds(d * L, L)] = jnp.zeros((L,), jnp.float32)
            return carry
        lax.fori_loop(0, K, zrow, 0)
        for z in range(RPT // K):
            pltpu.sync_copy(gath[0], acc_sh.at[pl.ds(base_row + z * K, K)])
        nfull = (RPT // K) * K

        @pl.when(sid < NS - 1)
        def _():
            pltpu.sync_copy(gath[0].at[pl.ds(0, RPT - nfull)],
                            acc_sh.at[pl.ds(base_row + nfull, RPT - nfull)])

        @pl.when(sid == NS - 1)
        def _():
            pltpu.sync_copy(gath[0].at[pl.ds(0, RPT_LAST - nfull)],
                            acc_sh.at[pl.ds(base_row + nfull, RPT_LAST - nfull)])
        plsc.subcore_barrier()

        wid = cid * NS + sid
        ebase = wid * E_PER_W
        pltpu.sync_copy(col_hbm.at[pl.ds(ebase, E_PER_W)], cols_v)

        def start(kk, b):
            pltpu.async_copy(
                feat_hbm.at[cols_v.at[pl.ds(kk * K, K)]], gath[b], gsem[b])
            pltpu.async_copy(row_hbm.at[pl.ds(ebase + kk * K, K)],
                             rowb[b], rsem[b])
            pltpu.async_copy(val_hbm.at[pl.ds(ebase + kk * K, K)],
                             valb[b], vsem[b])

        def drain_scatter(b):
            pltpu.make_async_copy(gath[b], acc_sh.at[rowb[b]], ssem[b]).wait()

        def process(kk, b):
            pltpu.make_async_copy(row_hbm.at[pl.ds(ebase + kk * K, K)],
                                  rowb[b], rsem[b]).wait()
            pltpu.make_async_copy(val_hbm.at[pl.ds(ebase + kk * K, K)],
                                  valb[b], vsem[b]).wait()
            pltpu.make_async_copy(
                feat_hbm.at[cols_v.at[pl.ds(kk * K, K)]], gath[b],
                gsem[b]).wait()

            def scale(g, c2):
                v16 = valb[b][pl.ds(g * L, L)]
                for j in range(L):
                    v = v16[j]
                    r = g * L + j
                    for d in range(D // L):
                        sl = pl.ds(d * L, L)
                        gath[b][r, sl] = gath[b][r, sl] * v
                return c2
            lax.fori_loop(0, K // L, scale, 0)
            pltpu.async_copy(gath[b], acc_sh.at[rowb[b]], ssem[b], add=True)

        start(0, 0)
        start(1, 1)

        def chunk(kk, carry):
            for b in range(NBUF):
                @pl.when(kk % NBUF == b)
                def _(b=b):
                    process(kk, b)
                    bn = (b + 2) % NBUF

                    @pl.when(kk >= 1)
                    def _():
                        drain_scatter(bn)

                    @pl.when(kk + 2 < CHUNKS)
                    def _():
                        start(kk + 2, bn)
            return carry
        lax.fori_loop(0, CHUNKS, chunk, 0)
        drain_scatter((CHUNKS - 1) % NBUF)

        plsc.subcore_barrier()
        @pl.when(sid < NS - 1)
        def _():
            pltpu.sync_copy(acc_sh.at[pl.ds(base_row, RPT)],
                            out_hbm.at[cid, pl.ds(base_row, RPT)])

        @pl.when(sid == NS - 1)
        def _():
            pltpu.sync_copy(acc_sh.at[pl.ds(base_row, RPT_LAST)],
                            out_hbm.at[cid, pl.ds(base_row, RPT_LAST)])

    return k(features, rows, cols, values)


def _tc_sum(partials):
    RB = 400

    def body(p_ref, o_ref):
        o_ref[...] = p_ref[0] + p_ref[1]

    return pl.pallas_call(
        body,
        grid=(N // RB,),
        in_specs=[pl.BlockSpec((2, RB, D), lambda i: (0, i, 0))],
        out_specs=pl.BlockSpec((RB, D), lambda i: (i, 0)),
        out_shape=jax.ShapeDtypeStruct((N, D), jnp.float32),
    )(partials)


def kernel(features, laplacianMat_indices, laplacianMat_values, selfLoop):
    del selfLoop
    rows = laplacianMat_indices[0]
    cols = laplacianMat_indices[1]
    partials = _sc_partials(features, rows, cols, laplacianMat_values)
    return _tc_sum(partials)
